# Initial kernel scaffold; baseline (speedup 1.0000x reference)
import jax
import jax.numpy as jnp
from jax.experimental import pallas as pl


def kernel(seq_len, weight):
    raise NotImplementedError("write your pallas kernel here")


# speedup vs baseline: 2.2555x; 2.2555x over previous
import math

import jax
import jax.numpy as jnp
import numpy as np
from jax import lax
from jax.experimental import pallas as pl
from jax.experimental.pallas import tpu as pltpu
from jax.experimental.pallas import tpu_sc as plsc

S = 4096
NUM_BUCKETS = 32
MAX_DISTANCE = 4096
NW = 32
RPW = S // NW
SPAN = 4224
GW = 8224
LANES = 16


def _bucket_table() -> np.ndarray:
    g = np.arange(GW, dtype=np.int64)
    rows = []
    for c in range(8):
        d = np.clip(g + c - (S - 1), -(S - 1), S - 1)
        a = np.abs(d)
        safe = np.maximum(a, 1).astype(np.float32)
        log_term = 8.0 + np.ceil(
            np.log(safe / 8.0) / math.log(MAX_DISTANCE / 8.0) * 8.0
        )
        large = np.minimum(np.float32(15.0), log_term).astype(np.int32)
        b = np.where(a < 8, a, large).astype(np.int32)
        rows.append(np.where(d < 0, b + 16, b).astype(np.int32))
    return np.stack(rows)


_BT = _bucket_table()


def _rpe_body(bt_hbm, w_hbm, out_hbm, *scratch):
    bt_v = scratch[0:8]
    v_v = scratch[8:16]
    w_v, sem = scratch[16], scratch[17]
    cid = lax.axis_index("c")
    sid = lax.axis_index("s")
    wid = sid * 2 + cid
    r0 = wid * RPW
    gbase = (S - RPW) - r0

    pltpu.sync_copy(w_hbm, w_v)
    for c in range(8):
        pltpu.sync_copy(bt_hbm.at[pl.ds(c * GW + gbase, SPAN)], bt_v[c])

    def gather_chunk(k, carry):
        for c in range(8):
            idx = bt_v[c][pl.ds(k * LANES, LANES)]
            v_v[c][pl.ds(k * LANES, LANES)] = plsc.load_gather(w_v, [idx])
        return carry

    lax.fori_loop(0, SPAN // LANES, gather_chunk, 0)

    def row_block(m, carry):
        o8 = (RPW - 8) - 8 * m
        handles = [
            pltpu.async_copy(
                v_v[c].at[pl.ds(o8, S)],
                out_hbm.at[pl.ds((r0 + (7 - c) + 8 * m) * S, S)],
                sem,
            )
            for c in range(8)
        ]
        for h in handles:
            h.wait()
        return carry

    lax.fori_loop(0, RPW // 8, row_block, 0)


def kernel(seq_len, weight):
    del seq_len
    w = weight.reshape(NUM_BUCKETS).astype(jnp.float32)
    run = pl.kernel(
        _rpe_body,
        out_type=jax.ShapeDtypeStruct((S * S,), jnp.float32),
        mesh=plsc.VectorSubcoreMesh(core_axis_name="c", subcore_axis_name="s"),
        compiler_params=pltpu.CompilerParams(needs_layout_passes=False),
        scratch_types=(
            [pltpu.VMEM((SPAN,), jnp.int32) for _ in range(8)]
            + [pltpu.VMEM((SPAN,), jnp.float32) for _ in range(8)]
            + [pltpu.VMEM((NUM_BUCKETS,), jnp.float32),
               pltpu.SemaphoreType.DMA]
        ),
    )
    return run(jnp.asarray(_BT).reshape(-1), w).reshape(S, S)
